# baseline (speedup 1.0000x reference)
import jax
import jax.numpy as jnp
from jax.experimental import pallas as pl


BLK_S = 1024


def _add_kernel(x_ref, pos_ref, o_ref):
    o_ref[...] = x_ref[...] + pos_ref[...][None, :, :]


def kernel(x, pos_table):
    B, S, D = x.shape
    grid = (S // BLK_S, B // 2)
    return pl.pallas_call(
        _add_kernel,
        grid=grid,
        in_specs=[
            pl.BlockSpec((2, BLK_S, D), lambda i, b: (b, i, 0)),
            pl.BlockSpec((BLK_S, D), lambda i, b: (i, 0)),
        ],
        out_specs=pl.BlockSpec((2, BLK_S, D), lambda i, b: (b, i, 0)),
        out_shape=jax.ShapeDtypeStruct((B, S, D), x.dtype),
    )(x, pos_table)


# speedup vs baseline: 1.0003x; 1.0003x over previous
import jax
import jax.numpy as jnp
from jax import lax
from jax.experimental import pallas as pl
from jax.experimental.pallas import tpu as pltpu


CH = 1024
K = 3


def _stream_kernel(x_hbm, pos_hbm, o_hbm, xbuf, pbuf, obuf, sx, sp, so):
    B, S, D = x_hbm.shape
    n_groups = S // CH
    T = n_groups * B

    def x_copy(t, slot):
        b = t % B
        i = t // B
        return pltpu.make_async_copy(
            x_hbm.at[b, pl.ds(i * CH, CH), :], xbuf.at[slot], sx.at[slot])

    def pos_copy(i):
        return pltpu.make_async_copy(
            pos_hbm.at[pl.ds(i * CH, CH), :], pbuf.at[i % 2], sp.at[i % 2])

    def out_copy(t, slot):
        b = t % B
        i = t // B
        return pltpu.make_async_copy(
            obuf.at[slot], o_hbm.at[b, pl.ds(i * CH, CH), :], so.at[slot])

    pos_copy(0).start()
    for t in range(K):
        x_copy(t, t).start()

    def step(t, _):
        s = lax.rem(t, K)
        b = lax.rem(t, B)
        i = lax.div(t, B)

        @pl.when(t >= K)
        def _():
            out_copy(t - K, s).wait()

        x_copy(t, s).wait()

        @pl.when(b == 0)
        def _():
            pos_copy(i).wait()

        obuf[s] = xbuf[s] + pbuf[lax.rem(i, 2)]
        out_copy(t, s).start()

        t2 = t + K
        @pl.when(t2 < T)
        def _():
            x_copy(t2, s).start()
            b2 = lax.rem(t2, B)
            @pl.when(b2 == 0)
            def _():
                pos_copy(lax.div(t2, B)).start()

        return ()

    lax.fori_loop(0, T, step, (), unroll=False)

    for dt in range(K):
        t = T - K + dt
        out_copy(t, t % K).wait()


def kernel(x, pos_table):
    B, S, D = x.shape
    return pl.pallas_call(
        _stream_kernel,
        in_specs=[
            pl.BlockSpec(memory_space=pl.ANY),
            pl.BlockSpec(memory_space=pl.ANY),
        ],
        out_specs=pl.BlockSpec(memory_space=pl.ANY),
        out_shape=jax.ShapeDtypeStruct((B, S, D), x.dtype),
        scratch_shapes=[
            pltpu.VMEM((K, CH, D), x.dtype),
            pltpu.VMEM((2, CH, D), x.dtype),
            pltpu.VMEM((K, CH, D), x.dtype),
            pltpu.SemaphoreType.DMA((K,)),
            pltpu.SemaphoreType.DMA((2,)),
            pltpu.SemaphoreType.DMA((K,)),
        ],
    )(x, pos_table)


# speedup vs baseline: 1.0059x; 1.0056x over previous
import jax
import jax.numpy as jnp
from jax.experimental import pallas as pl


BLK_S = 2048


def _add_kernel(x_ref, pos_ref, o_ref):
    o_ref[...] = x_ref[...] + pos_ref[...][None, :, :]


def kernel(x, pos_table):
    B, S, D = x.shape
    grid = (S // BLK_S, B)
    return pl.pallas_call(
        _add_kernel,
        grid=grid,
        in_specs=[
            pl.BlockSpec((1, BLK_S, D), lambda i, b: (b, i, 0)),
            pl.BlockSpec((BLK_S, D), lambda i, b: (i, 0)),
        ],
        out_specs=pl.BlockSpec((1, BLK_S, D), lambda i, b: (b, i, 0)),
        out_shape=jax.ShapeDtypeStruct((B, S, D), x.dtype),
    )(x, pos_table)
